# Initial kernel scaffold; baseline (speedup 1.0000x reference)
import jax
import jax.numpy as jnp
from jax.experimental import pallas as pl


def kernel(uv, layer1, layer2, layer3, layer4):
    raise NotImplementedError("write your pallas kernel here")


# speedup vs baseline: 1.2405x; 1.2405x over previous
import functools

import jax
import jax.numpy as jnp
from jax import lax
from jax.experimental import pallas as pl
from jax.experimental.pallas import tpu as pltpu
from jax.experimental.pallas import tpu_sc as plsc

_B, _HOUT, _WOUT = 4, 512, 512
_NS = _B * _HOUT * _WOUT
_NW = 32
_PW = _NS // _NW
_S = 2048
_NCHUNK = _PW // _S
_SB = _S // 128
_LEVELS = (4096, 2048, 1024, 512)


def _tec_body(u_hbm, v_hbm, t1, t2, t3, t4, out_hbm,
              u_v, v_v, idx_refs, val_refs, w_v, o_v, sem):
    tables = (t1, t2, t3, t4)
    wid = lax.axis_index("s") * 2 + lax.axis_index("c")
    wbase = wid * _PW

    @pl.loop(0, _NCHUNK)
    def chunk(c):
        base = wbase + c * _S
        pltpu.sync_copy(u_hbm.at[pl.ds(base, _S)], u_v)
        pltpu.sync_copy(v_hbm.at[pl.ds(base, _S)], v_v)

        @pl.loop(0, _S // 16)
        def compute(j):
            off = j * 16
            u = u_v[pl.ds(off, 16)]
            v = v_v[pl.ds(off, 16)]
            gx = u * 2.0 - 1.0
            gy = v * 2.0 - 1.0
            for l, n in enumerate(_LEVELS):
                nf = float(n)
                ix = ((gx + 1.0) * nf - 1.0) / 2.0
                iy = ((gy + 1.0) * nf - 1.0) / 2.0
                x0p1 = (ix + 1.0).astype(jnp.int32)
                y0p1 = (iy + 1.0).astype(jnp.int32)
                ix0 = x0p1 - 1
                iy0 = y0p1 - 1
                wx1 = ix - ix0.astype(jnp.float32)
                wy1 = iy - iy0.astype(jnp.float32)
                wx0 = 1.0 - wx1
                wy0 = 1.0 - wy1
                vx0 = ix0 >= 0
                vy0 = iy0 >= 0
                vx1 = x0p1 <= n - 1
                vy1 = y0p1 <= n - 1
                xc0 = jnp.maximum(ix0, 0)
                yc0 = jnp.maximum(iy0, 0)
                xc1 = jnp.minimum(x0p1, n - 1)
                yc1 = jnp.minimum(y0p1, n - 1)
                wx0 = jnp.where(vx0, wx0, 0.0)
                wx1 = jnp.where(vx1, wx1, 0.0)
                wy0 = jnp.where(vy0, wy0, 0.0)
                wy1 = jnp.where(vy1, wy1, 0.0)
                b0 = yc0 * n
                b1 = yc1 * n
                t = 4 * l
                idx_refs[t + 0][pl.ds(off, 16)] = b0 + xc0
                idx_refs[t + 1][pl.ds(off, 16)] = b0 + xc1
                idx_refs[t + 2][pl.ds(off, 16)] = b1 + xc0
                idx_refs[t + 3][pl.ds(off, 16)] = b1 + xc1
                w_v[t + 0, pl.ds(off, 16)] = wy0 * wx0
                w_v[t + 1, pl.ds(off, 16)] = wy0 * wx1
                w_v[t + 2, pl.ds(off, 16)] = wy1 * wx0
                w_v[t + 3, pl.ds(off, 16)] = wy1 * wx1

        copies = [
            pltpu.async_copy(tables[t // 4].at[idx_refs[t]], val_refs[t], sem)
            for t in range(16)
        ]
        for cp in copies:
            cp.wait()

        @pl.loop(0, _S // 16)
        def combine(j):
            off = j * 16
            acc = val_refs[0][pl.ds(off, 16)] * w_v[0, pl.ds(off, 16)]
            for t in range(1, 16):
                acc = acc + val_refs[t][pl.ds(off, 16)] * w_v[t, pl.ds(off, 16)]
            o_v[pl.ds(off, 16)] = acc

        pltpu.sync_copy(o_v, out_hbm.at[pl.ds(base, _S)])


@functools.partial(
    pl.kernel,
    out_type=jax.ShapeDtypeStruct((_NS,), jnp.float32),
    mesh=plsc.VectorSubcoreMesh(core_axis_name="c", subcore_axis_name="s"),
    scratch_types=[
        pltpu.VMEM((_S,), jnp.float32),
        pltpu.VMEM((_S,), jnp.float32),
        [pltpu.VMEM((_S,), jnp.int32)] * 16,
        [pltpu.VMEM((_S,), jnp.float32)] * 16,
        pltpu.VMEM((16, _S), jnp.float32),
        pltpu.VMEM((_S,), jnp.float32),
        pltpu.SemaphoreType.DMA,
    ],
)
def _sc_sample(u_hbm, v_hbm, t1, t2, t3, t4, out_hbm,
               u_v, v_v, idx_refs, val_refs, w_v, o_v, sem):
    _tec_body(u_hbm, v_hbm, t1, t2, t3, t4, out_hbm,
              u_v, v_v, idx_refs, val_refs, w_v, o_v, sem)


@jax.jit
def kernel(uv, layer1, layer2, layer3, layer4):
    uvf = uv.reshape(_NS, 2)
    u = uvf[:, 0]
    v = uvf[:, 1]
    out = _sc_sample(
        u, v,
        layer1.reshape(-1), layer2.reshape(-1),
        layer3.reshape(-1), layer4.reshape(-1),
    )
    return out.reshape(_B, 1, _HOUT, _WOUT)


# speedup vs baseline: 1.4544x; 1.1724x over previous
import functools

import jax
import jax.numpy as jnp
from jax import lax
from jax.experimental import pallas as pl
from jax.experimental.pallas import tpu as pltpu
from jax.experimental.pallas import tpu_sc as plsc

_B, _HOUT, _WOUT = 4, 512, 512
_NS = _B * _HOUT * _WOUT
_NW = 32
_PW = _NS // _NW
_S = 1024
_NCHUNK = _PW // _S
_LEVELS = (4096, 2048, 1024, 512)


def _tec_body(u_hbm, v_hbm, t1, t2, t3, t4, out_hbm,
              u_v, v_v, idx_a, val_a, w_a, idx_b, val_b, w_b,
              o_v, sem_a, sem_b):
    tables = (t1, t2, t3, t4)
    bufs = ((idx_a, val_a, w_a, sem_a), (idx_b, val_b, w_b, sem_b))
    wid = lax.axis_index("s") * 2 + lax.axis_index("c")
    wbase = wid * _PW

    def load_uv(c):
        base = wbase + c * _S
        pltpu.sync_copy(u_hbm.at[pl.ds(base, _S)], u_v)
        pltpu.sync_copy(v_hbm.at[pl.ds(base, _S)], v_v)

    def compute(p):
        idx_refs, _, w_v, _ = bufs[p]

        @pl.loop(0, _S // 16)
        def _(j):
            off = j * 16
            u = u_v[pl.ds(off, 16)]
            v = v_v[pl.ds(off, 16)]
            gx = u * 2.0 - 1.0
            gy = v * 2.0 - 1.0
            for l, n in enumerate(_LEVELS):
                nf = float(n)
                ix = ((gx + 1.0) * nf - 1.0) / 2.0
                iy = ((gy + 1.0) * nf - 1.0) / 2.0
                x0p1 = (ix + 1.0).astype(jnp.int32)
                y0p1 = (iy + 1.0).astype(jnp.int32)
                ix0 = x0p1 - 1
                iy0 = y0p1 - 1
                wx1 = ix - ix0.astype(jnp.float32)
                wy1 = iy - iy0.astype(jnp.float32)
                wx0 = 1.0 - wx1
                wy0 = 1.0 - wy1
                vx0 = ix0 >= 0
                vy0 = iy0 >= 0
                vx1 = x0p1 <= n - 1
                vy1 = y0p1 <= n - 1
                xc0 = jnp.maximum(ix0, 0)
                yc0 = jnp.maximum(iy0, 0)
                xc1 = jnp.minimum(x0p1, n - 1)
                yc1 = jnp.minimum(y0p1, n - 1)
                wx0 = jnp.where(vx0, wx0, 0.0)
                wx1 = jnp.where(vx1, wx1, 0.0)
                wy0 = jnp.where(vy0, wy0, 0.0)
                wy1 = jnp.where(vy1, wy1, 0.0)
                b0 = yc0 * n
                b1 = yc1 * n
                t = 4 * l
                idx_refs[t + 0][pl.ds(off, 16)] = b0 + xc0
                idx_refs[t + 1][pl.ds(off, 16)] = b0 + xc1
                idx_refs[t + 2][pl.ds(off, 16)] = b1 + xc0
                idx_refs[t + 3][pl.ds(off, 16)] = b1 + xc1
                w_v[t + 0, pl.ds(off, 16)] = wy0 * wx0
                w_v[t + 1, pl.ds(off, 16)] = wy0 * wx1
                w_v[t + 2, pl.ds(off, 16)] = wy1 * wx0
                w_v[t + 3, pl.ds(off, 16)] = wy1 * wx1

    def fire(p):
        idx_refs, val_refs, _, sem = bufs[p]
        for t in range(16):
            pltpu.async_copy(tables[t // 4].at[idx_refs[t]], val_refs[t], sem)

    def drain(p):
        idx_refs, val_refs, _, sem = bufs[p]
        for t in range(16):
            pltpu.make_async_copy(
                tables[t // 4].at[idx_refs[t]], val_refs[t], sem).wait()

    def combine_store(c, p):
        _, val_refs, w_v, _ = bufs[p]

        @pl.loop(0, _S // 16)
        def _(j):
            off = j * 16
            acc = val_refs[0][pl.ds(off, 16)] * w_v[0, pl.ds(off, 16)]
            for t in range(1, 16):
                acc = acc + val_refs[t][pl.ds(off, 16)] * w_v[t, pl.ds(off, 16)]
            o_v[pl.ds(off, 16)] = acc

        pltpu.sync_copy(o_v, out_hbm.at[pl.ds(wbase + c * _S, _S)])

    load_uv(0)
    compute(0)
    fire(0)

    @pl.loop(0, _NCHUNK // 2)
    def _(tt):
        i = tt * 2 + 1
        load_uv(i)
        compute(1)
        fire(1)
        drain(0)
        combine_store(i - 1, 0)

        @pl.when(tt < _NCHUNK // 2 - 1)
        def _():
            i2 = i + 1
            load_uv(i2)
            compute(0)
            fire(0)
            drain(1)
            combine_store(i2 - 1, 1)

    drain(1)
    combine_store(_NCHUNK - 1, 1)


@functools.partial(
    pl.kernel,
    out_type=jax.ShapeDtypeStruct((_NS,), jnp.float32),
    mesh=plsc.VectorSubcoreMesh(core_axis_name="c", subcore_axis_name="s"),
    scratch_types=[
        pltpu.VMEM((_S,), jnp.float32),
        pltpu.VMEM((_S,), jnp.float32),
        [pltpu.VMEM((_S,), jnp.int32)] * 16,
        [pltpu.VMEM((_S,), jnp.float32)] * 16,
        pltpu.VMEM((16, _S), jnp.float32),
        [pltpu.VMEM((_S,), jnp.int32)] * 16,
        [pltpu.VMEM((_S,), jnp.float32)] * 16,
        pltpu.VMEM((16, _S), jnp.float32),
        pltpu.VMEM((_S,), jnp.float32),
        pltpu.SemaphoreType.DMA,
        pltpu.SemaphoreType.DMA,
    ],
)
def _sc_sample(u_hbm, v_hbm, t1, t2, t3, t4, out_hbm,
               u_v, v_v, idx_a, val_a, w_a, idx_b, val_b, w_b,
               o_v, sem_a, sem_b):
    _tec_body(u_hbm, v_hbm, t1, t2, t3, t4, out_hbm,
              u_v, v_v, idx_a, val_a, w_a, idx_b, val_b, w_b,
              o_v, sem_a, sem_b)


@jax.jit
def kernel(uv, layer1, layer2, layer3, layer4):
    uvf = uv.reshape(_NS, 2)
    u = uvf[:, 0]
    v = uvf[:, 1]
    out = _sc_sample(
        u, v,
        layer1.reshape(-1), layer2.reshape(-1),
        layer3.reshape(-1), layer4.reshape(-1),
    )
    return out.reshape(_B, 1, _HOUT, _WOUT)
